# Initial kernel scaffold; baseline (speedup 1.0000x reference)
import jax
import jax.numpy as jnp
from jax.experimental import pallas as pl


def kernel(x, emb, W1, b1, W2, b2):
    raise NotImplementedError("write your pallas kernel here")


# speedup vs baseline: 2.8668x; 2.8668x over previous
import functools

import jax
import jax.numpy as jnp
from jax import lax
from jax.experimental import pallas as pl
from jax.experimental.pallas import tpu as pltpu
from jax.experimental.pallas import tpu_sc as plsc

B = 16384
S = 200
D = 64
HIDDEN = 128
NUM_CLASSES = 100

NC = 2
NS = 16
NW = NC * NS
NB = B // NW
CE = 64
NCHUNK = NB // CE
HALF = 100
INV_S = 1.0 / S


def _pool_body(x_hbm, emb_hbm, h_hbm, idxv, buf0, buf1, hbuf, sem0, sem1):
    wid = lax.axis_index("s") * NC + lax.axis_index("c")
    base = wid * NB

    bufs = (buf0, buf1)
    sems = (sem0, sem1)

    def issue(e_local, buf, sem):
        pltpu.async_copy(emb_hbm.at[idxv.at[2 * e_local]],
                         buf.at[pl.ds(0, HALF)], sem)
        pltpu.async_copy(emb_hbm.at[idxv.at[2 * e_local + 1]],
                         buf.at[pl.ds(HALF, HALF)], sem)

    def wait(buf, sem):
        pltpu.make_async_copy(emb_hbm.at[pl.ds(0, S)], buf, sem).wait()

    def accumulate(buf, bidx):
        def rbody(r, accs):
            return tuple(accs[c] + buf[r, pl.ds(16 * c, 16)]
                         for c in range(4))
        accs = lax.fori_loop(
            0, S, rbody,
            tuple(jnp.zeros((16,), jnp.float32) for _ in range(4)),
            unroll=8)
        for c in range(4):
            hbuf[bidx, pl.ds(16 * c, 16)] = accs[c] * INV_S

    def chunk_body(c, _):
        xrow = (base + c * CE) * 2
        pltpu.sync_copy(x_hbm.at[pl.ds(xrow, 2 * CE)], idxv)
        issue(0, bufs[0], sems[0])

        def ebody(j, _):
            e0 = 2 * j
            issue(e0 + 1, bufs[1], sems[1])
            wait(bufs[0], sems[0])
            accumulate(bufs[0], c * CE + e0)

            @pl.when(j < CE // 2 - 1)
            def _():
                issue(e0 + 2, bufs[0], sems[0])

            wait(bufs[1], sems[1])
            accumulate(bufs[1], c * CE + e0 + 1)
            return 0

        lax.fori_loop(0, CE // 2, ebody, 0)
        return 0

    lax.fori_loop(0, NCHUNK, chunk_body, 0)
    pltpu.sync_copy(hbuf, h_hbm.at[pl.ds(base, NB)])


@jax.jit
def _sc_pool(x2, emb):
    mesh = plsc.VectorSubcoreMesh(core_axis_name="c", subcore_axis_name="s")
    return pl.kernel(
        _pool_body,
        out_type=jax.ShapeDtypeStruct((B, D), jnp.float32),
        mesh=mesh,
        scratch_types=[
            pltpu.VMEM((2 * CE, HALF), jnp.int32),
            pltpu.VMEM((S, D), jnp.float32),
            pltpu.VMEM((S, D), jnp.float32),
            pltpu.VMEM((NB, D), jnp.float32),
            pltpu.SemaphoreType.DMA,
            pltpu.SemaphoreType.DMA,
        ],
        compiler_params=pltpu.CompilerParams(use_tc_tiling_on_sc=False),
    )(x2, emb)


def _mlp_body(h_ref, w1_ref, b1_ref, w2_ref, b2_ref, o_ref):
    h1 = jnp.dot(h_ref[...], w1_ref[...],
                 preferred_element_type=jnp.float32) + b1_ref[...]
    h1 = jnp.maximum(h1, 0.0)
    o_ref[...] = jnp.dot(h1, w2_ref[...],
                         preferred_element_type=jnp.float32) + b2_ref[...]


@jax.jit
def _tc_mlp(h, W1, b1, W2, b2):
    BM = 2048
    grid = (B // BM,)
    return pl.pallas_call(
        _mlp_body,
        grid=grid,
        in_specs=[
            pl.BlockSpec((BM, D), lambda i: (i, 0)),
            pl.BlockSpec((D, HIDDEN), lambda i: (0, 0)),
            pl.BlockSpec((1, HIDDEN), lambda i: (0, 0)),
            pl.BlockSpec((HIDDEN, NUM_CLASSES), lambda i: (0, 0)),
            pl.BlockSpec((1, NUM_CLASSES), lambda i: (0, 0)),
        ],
        out_specs=pl.BlockSpec((BM, NUM_CLASSES), lambda i: (i, 0)),
        out_shape=jax.ShapeDtypeStruct((B, NUM_CLASSES), jnp.float32),
    )(h, W1, b1.reshape(1, HIDDEN), W2, b2.reshape(1, NUM_CLASSES))


def kernel(x, emb, W1, b1, W2, b2):
    x2 = x.reshape(B * S // HALF, HALF)
    h = _sc_pool(x2, emb)
    return _tc_mlp(h, W1, b1, W2, b2)


# speedup vs baseline: 2.9174x; 1.0176x over previous
import functools

import jax
import jax.numpy as jnp
from jax import lax
from jax.experimental import pallas as pl
from jax.experimental.pallas import tpu as pltpu
from jax.experimental.pallas import tpu_sc as plsc

B = 16384
S = 200
D = 64
HIDDEN = 128
NUM_CLASSES = 100

NC = 2
NS = 16
NW = NC * NS
NB = B // NW
CE = 64
NCHUNK = NB // CE
SPLIT = 96
INV_S = 1.0 / S


def _pool_body(x_hbm, emb_hbm, h_hbm, idxv, buf0, buf1, hbuf, sem0, sem1):
    wid = lax.axis_index("s") * NC + lax.axis_index("c")
    base = wid * NB

    bufs = (buf0, buf1)
    sems = (sem0, sem1)

    def issue(e_local, buf, sem):
        pltpu.async_copy(emb_hbm.at[idxv.at[e_local, pl.ds(0, SPLIT)]],
                         buf.at[pl.ds(0, SPLIT)], sem)
        pltpu.async_copy(emb_hbm.at[idxv.at[e_local, pl.ds(SPLIT, S - SPLIT)]],
                         buf.at[pl.ds(SPLIT, S - SPLIT)], sem)

    def wait(buf, sem):
        pltpu.make_async_copy(emb_hbm.at[pl.ds(0, S)], buf, sem).wait()

    def accumulate(buf, bidx):
        def rbody(r, accs):
            return tuple(accs[c] + buf[r, pl.ds(16 * c, 16)]
                         for c in range(4))
        accs = lax.fori_loop(
            0, S, rbody,
            tuple(jnp.zeros((16,), jnp.float32) for _ in range(4)),
            unroll=8)
        for c in range(4):
            hbuf[bidx, pl.ds(16 * c, 16)] = accs[c] * INV_S

    def chunk_body(c, _):
        xrow = base + c * CE
        pltpu.sync_copy(x_hbm.at[pl.ds(xrow, CE)], idxv)
        issue(0, bufs[0], sems[0])

        def ebody(j, _):
            e0 = 2 * j
            issue(e0 + 1, bufs[1], sems[1])
            wait(bufs[0], sems[0])
            accumulate(bufs[0], c * CE + e0)

            @pl.when(j < CE // 2 - 1)
            def _():
                issue(e0 + 2, bufs[0], sems[0])

            wait(bufs[1], sems[1])
            accumulate(bufs[1], c * CE + e0 + 1)
            return 0

        lax.fori_loop(0, CE // 2, ebody, 0)
        return 0

    lax.fori_loop(0, NCHUNK, chunk_body, 0)
    pltpu.sync_copy(hbuf, h_hbm.at[pl.ds(base, NB)])


@jax.jit
def _sc_pool(x, emb):
    mesh = plsc.VectorSubcoreMesh(core_axis_name="c", subcore_axis_name="s")
    return pl.kernel(
        _pool_body,
        out_type=jax.ShapeDtypeStruct((B, D), jnp.float32),
        mesh=mesh,
        scratch_types=[
            pltpu.VMEM((CE, S), jnp.int32),
            pltpu.VMEM((S, D), jnp.float32),
            pltpu.VMEM((S, D), jnp.float32),
            pltpu.VMEM((NB, D), jnp.float32),
            pltpu.SemaphoreType.DMA,
            pltpu.SemaphoreType.DMA,
        ],
        compiler_params=pltpu.CompilerParams(use_tc_tiling_on_sc=False),
    )(x, emb)


def _mlp_body(h_ref, w1_ref, b1_ref, w2_ref, b2_ref, o_ref):
    h1 = jnp.dot(h_ref[...], w1_ref[...],
                 preferred_element_type=jnp.float32) + b1_ref[...]
    h1 = jnp.maximum(h1, 0.0)
    o_ref[...] = jnp.dot(h1, w2_ref[...],
                         preferred_element_type=jnp.float32) + b2_ref[...]


@jax.jit
def _tc_mlp(h, W1, b1, W2, b2):
    BM = 2048
    grid = (B // BM,)
    return pl.pallas_call(
        _mlp_body,
        grid=grid,
        in_specs=[
            pl.BlockSpec((BM, D), lambda i: (i, 0)),
            pl.BlockSpec((D, HIDDEN), lambda i: (0, 0)),
            pl.BlockSpec((1, HIDDEN), lambda i: (0, 0)),
            pl.BlockSpec((HIDDEN, NUM_CLASSES), lambda i: (0, 0)),
            pl.BlockSpec((1, NUM_CLASSES), lambda i: (0, 0)),
        ],
        out_specs=pl.BlockSpec((BM, NUM_CLASSES), lambda i: (i, 0)),
        out_shape=jax.ShapeDtypeStruct((B, NUM_CLASSES), jnp.float32),
    )(h, W1, b1.reshape(1, HIDDEN), W2, b2.reshape(1, NUM_CLASSES))


def kernel(x, emb, W1, b1, W2, b2):
    h = _sc_pool(x, emb)
    return _tc_mlp(h, W1, b1, W2, b2)
